# baseline (speedup 1.0000x reference)
import jax
import jax.numpy as jnp
from jax.experimental import pallas as pl
from jax.experimental.pallas import tpu as pltpu

NUM_CLASSES = 21
THRESHOLD = 0.5
NEGPOS_RATIO = 3
VAR0 = 0.1
VAR1 = 0.2
EPS = 0.05
CLIP_LO = -16.11809565095832
CLIP_HI = -1.0000000494736474e-07


def _phase_a(conf_ref, loc_ref, pri_ref, tgt_ref, ml_ref, rl_ref, st_ref):
    conf = jnp.transpose(conf_ref[0], (1, 0))
    locd = jnp.transpose(loc_ref[0], (1, 0))
    pri = pri_ref[...]
    tgt = tgt_ref[0]

    P = conf.shape[1]
    T = tgt.shape[0]

    cx = pri[0:1, :]
    cy = pri[1:2, :]
    w = pri[2:3, :]
    h = pri[3:4, :]
    pxmin = cx - w * 0.5
    pymin = cy - h * 0.5
    pxmax = cx + w * 0.5
    pymax = cy + h * 0.5

    txmin = tgt[:, 0:1]
    tymin = tgt[:, 1:2]
    txmax = tgt[:, 2:3]
    tymax = tgt[:, 3:4]
    tlab = tgt[:, 4:5]

    iw = jnp.clip(jnp.minimum(txmax, pxmax) - jnp.maximum(txmin, pxmin), 0.0, None)
    ih = jnp.clip(jnp.minimum(tymax, pymax) - jnp.maximum(tymin, pymin), 0.0, None)
    inter = iw * ih
    area_a = (txmax - txmin) * (tymax - tymin)
    area_b = (pxmax - pxmin) * (pymax - pymin)
    ov = inter / (area_a + area_b - inter)

    sub = jax.lax.broadcasted_iota(jnp.int32, (T, P), 0)
    bto = jnp.max(ov, axis=0, keepdims=True)
    bti = jnp.min(jnp.where(ov == bto, sub, T), axis=0, keepdims=True)

    lane = jax.lax.broadcasted_iota(jnp.int32, (T, P), 1)
    rmax = jnp.max(ov, axis=1, keepdims=True)
    bpi = jnp.min(jnp.where(ov == rmax, lane, P), axis=1, keepdims=True)

    hit = bpi == lane
    forced_t = jnp.max(jnp.where(hit, sub, -1), axis=0, keepdims=True)
    forced = forced_t >= 0
    bti = jnp.where(forced, forced_t, bti)
    bto = jnp.where(forced, 2.0, bto)

    eq_f = (sub == bti).astype(jnp.float32)
    matched = jax.lax.dot_general(
        tgt, eq_f, (((0,), (0,)), ((), ())),
        preferred_element_type=jnp.float32)
    mxmin = matched[0:1, :]
    mymin = matched[1:2, :]
    mxmax = matched[2:3, :]
    mymax = matched[3:4, :]
    mlab = matched[4:5, :]

    conf_t = jnp.where(bto < THRESHOLD, 0, mlab.astype(jnp.int32) + 1)
    pos = conf_t > 0

    g0 = ((mxmin + mxmax) * 0.5 - cx) / (VAR0 * w)
    g1 = ((mymin + mymax) * 0.5 - cy) / (VAR0 * h)
    g2 = jnp.log((mxmax - mxmin) / w) / VAR1
    g3 = jnp.log((mymax - mymin) / h) / VAR1
    sl1 = jnp.zeros((1, P), jnp.float32)
    for c, g in enumerate((g0, g1, g2, g3)):
        d = locd[c:c + 1, :] - g
        a = jnp.abs(d)
        sl1 = sl1 + jnp.where(a < 1.0, 0.5 * d * d, a - 0.5)
    loss_l_2d = jnp.sum(jnp.where(pos, sl1, 0.0), axis=1, keepdims=True)

    ones_c = jnp.ones((1, NUM_CLASSES), jnp.float32)
    m = jnp.max(conf, axis=0, keepdims=True)
    e = jnp.exp(conf - m)
    s = jax.lax.dot_general(
        ones_c, e, (((1,), (0,)), ((), ())),
        preferred_element_type=jnp.float32)
    mls = m + jnp.log(s)
    lp = jnp.clip(conf - mls, CLIP_LO, CLIP_HI)
    csub = jax.lax.broadcasted_iota(jnp.int32, (NUM_CLASSES, P), 0)
    eq_c = csub == conf_t
    gathered = jnp.sum(jnp.where(eq_c, conf, 0.0), axis=0, keepdims=True)
    lpu_tgt = gathered - mls
    lp_tgt = jnp.clip(lpu_tgt, CLIP_LO, CLIP_HI)
    lp_all = jax.lax.dot_general(
        ones_c, lp, (((1,), (0,)), ((), ())),
        preferred_element_type=jnp.float32)
    mloss = jnp.where(pos, 0.0, -lpu_tgt)
    eps_o = EPS / (NUM_CLASSES - 1)
    row_loss = -(eps_o * lp_all + (1.0 - EPS - eps_o) * lp_tgt)
    rl_neg = jnp.where(pos, 0.0, row_loss)
    rl_pos_2d = jnp.sum(jnp.where(pos, row_loss, 0.0), axis=1, keepdims=True)
    npos_2d = jnp.sum(pos.astype(jnp.float32), axis=1, keepdims=True)

    ml_ref[...] = mloss.reshape(1, 1, P)
    rl_ref[...] = rl_neg.reshape(1, 1, P)
    li = jax.lax.broadcasted_iota(jnp.int32, (1, 128), 1)
    stats = (jnp.where(li == 0, loss_l_2d, 0.0)
             + jnp.where(li == 1, npos_2d, 0.0)
             + jnp.where(li == 2, rl_pos_2d, 0.0))
    st_ref[...] = stats.reshape(1, 1, 128)


def _phase_b(ml_ref, rl_ref, st_ref, out_l, out_c):
    ml = ml_ref[...][:, 0, :]
    rl = rl_ref[...][:, 0, :]
    st = st_ref[...][:, 0, :]
    B, P = ml.shape

    ll_tot = jnp.sum(st[:, 0:1], axis=0, keepdims=True)
    npos = st[:, 1:2]
    rp_tot = jnp.sum(st[:, 2:3], axis=0, keepdims=True)
    n_tot = jnp.maximum(jnp.sum(npos, axis=0, keepdims=True), 1.0)

    k = jnp.minimum(NEGPOS_RATIO * npos.astype(jnp.int32), P - 1)
    bits = jax.lax.bitcast_convert_type(ml, jnp.int32)

    def vstep(_, lh):
        lo, hi = lh
        mid = lo + (hi - lo) // 2
        cnt = jnp.sum(jnp.where(bits > mid, 1, 0), axis=1, keepdims=True)
        p = cnt < k
        return jnp.where(p, lo, mid + 1), jnp.where(p, mid, hi)

    zc = jnp.zeros((B, 1), jnp.int32)
    lo, hi = jax.lax.fori_loop(
        0, 31, vstep, (zc, jnp.full((B, 1), 0x7F800000, jnp.int32)))
    tb = hi
    gcnt = jnp.sum(jnp.where(bits > tb, 1, 0), axis=1, keepdims=True)
    need = k - gcnt
    equal = bits == tb
    lane = jax.lax.broadcasted_iota(jnp.int32, (B, P), 1)

    def istep(_, lh):
        lo2, hi2 = lh
        mid = lo2 + (hi2 - lo2) // 2
        cnt = jnp.sum(jnp.where(equal & (lane < mid), 1, 0),
                      axis=1, keepdims=True)
        p = cnt >= need
        return jnp.where(p, lo2, mid + 1), jnp.where(p, mid, hi2)

    j, _ = jax.lax.fori_loop(0, 14, istep, (zc, jnp.full((B, 1), P, jnp.int32)))
    neg = (bits > tb) | (equal & (lane < j))
    lc_neg = jnp.sum(jnp.sum(jnp.where(neg, rl, 0.0), axis=1, keepdims=True),
                     axis=0, keepdims=True)

    out_l[...] = ll_tot / n_tot
    out_c[...] = (rp_tot + lc_neg) / n_tot


def kernel(loc_data, conf_data, priors, targets):
    num, num_priors, _ = loc_data.shape
    pri_cm = jnp.transpose(priors[:num_priors], (1, 0))

    ml, rl, st = pl.pallas_call(
        _phase_a,
        grid=(num,),
        in_specs=[
            pl.BlockSpec((1, num_priors, NUM_CLASSES), lambda i: (i, 0, 0)),
            pl.BlockSpec((1, num_priors, 4), lambda i: (i, 0, 0)),
            pl.BlockSpec((4, num_priors), lambda i: (0, 0)),
            pl.BlockSpec((1, targets.shape[1], targets.shape[2]),
                         lambda i: (i, 0, 0)),
        ],
        out_specs=[
            pl.BlockSpec((1, 1, num_priors), lambda i: (i, 0, 0)),
            pl.BlockSpec((1, 1, num_priors), lambda i: (i, 0, 0)),
            pl.BlockSpec((1, 1, 128), lambda i: (i, 0, 0)),
        ],
        out_shape=[
            jax.ShapeDtypeStruct((num, 1, num_priors), jnp.float32),
            jax.ShapeDtypeStruct((num, 1, num_priors), jnp.float32),
            jax.ShapeDtypeStruct((num, 1, 128), jnp.float32),
        ],
        compiler_params=pltpu.CompilerParams(
            dimension_semantics=("parallel",)),
    )(conf_data, loc_data, pri_cm, targets)

    out_l, out_c = pl.pallas_call(
        _phase_b,
        out_shape=[jax.ShapeDtypeStruct((1, 1), jnp.float32)] * 2,
    )(ml, rl, st)
    return (out_l[0, 0], out_c[0, 0])


# speedup vs baseline: 1.7062x; 1.7062x over previous
import jax
import jax.numpy as jnp
from jax.experimental import pallas as pl
from jax.experimental.pallas import tpu as pltpu

NUM_CLASSES = 21
THRESHOLD = 0.5
NEGPOS_RATIO = 3
VAR0 = 0.1
VAR1 = 0.2
EPS = 0.05
CLIP_LO = -16.11809565095832
CLIP_HI = -1.0000000494736474e-07


def _phase_a(conf_ref, loc_ref, pri_ref, tgt_ref, ml_ref, rl_ref, st_ref):
    for img in range(conf_ref.shape[0]):
        _phase_a_one(conf_ref[img], loc_ref[img], pri_ref[...], tgt_ref[img],
                     ml_ref, rl_ref, st_ref, img)


def _phase_a_one(conf, locd, pri, tgt, ml_ref, rl_ref, st_ref, img):
    P = conf.shape[1]
    T = tgt.shape[0]

    cx = pri[0:1, :]
    cy = pri[1:2, :]
    w = pri[2:3, :]
    h = pri[3:4, :]
    pxmin = cx - w * 0.5
    pymin = cy - h * 0.5
    pxmax = cx + w * 0.5
    pymax = cy + h * 0.5

    txmin = tgt[:, 0:1]
    tymin = tgt[:, 1:2]
    txmax = tgt[:, 2:3]
    tymax = tgt[:, 3:4]
    tlab = tgt[:, 4:5]

    iw = jnp.clip(jnp.minimum(txmax, pxmax) - jnp.maximum(txmin, pxmin), 0.0, None)
    ih = jnp.clip(jnp.minimum(tymax, pymax) - jnp.maximum(tymin, pymin), 0.0, None)
    inter = iw * ih
    area_a = (txmax - txmin) * (tymax - tymin)
    area_b = (pxmax - pxmin) * (pymax - pymin)
    ov = inter / (area_a + area_b - inter)

    sub = jax.lax.broadcasted_iota(jnp.int32, (T, P), 0)
    bto = jnp.max(ov, axis=0, keepdims=True)
    bti = jnp.min(jnp.where(ov == bto, sub, T), axis=0, keepdims=True)

    lane = jax.lax.broadcasted_iota(jnp.int32, (T, P), 1)
    rmax = jnp.max(ov, axis=1, keepdims=True)
    bpi = jnp.min(jnp.where(ov == rmax, lane, P), axis=1, keepdims=True)

    hit = bpi == lane
    forced_t = jnp.max(jnp.where(hit, sub, -1), axis=0, keepdims=True)
    forced = forced_t >= 0
    bti = jnp.where(forced, forced_t, bti)
    bto = jnp.where(forced, 2.0, bto)

    eq_f = (sub == bti).astype(jnp.float32)
    matched = jax.lax.dot_general(
        tgt, eq_f, (((0,), (0,)), ((), ())),
        preferred_element_type=jnp.float32)
    mxmin = matched[0:1, :]
    mymin = matched[1:2, :]
    mxmax = matched[2:3, :]
    mymax = matched[3:4, :]
    mlab = matched[4:5, :]

    conf_t = jnp.where(bto < THRESHOLD, 0, mlab.astype(jnp.int32) + 1)
    pos = conf_t > 0

    g0 = ((mxmin + mxmax) * 0.5 - cx) / (VAR0 * w)
    g1 = ((mymin + mymax) * 0.5 - cy) / (VAR0 * h)
    g2 = jnp.log((mxmax - mxmin) / w) / VAR1
    g3 = jnp.log((mymax - mymin) / h) / VAR1
    sl1 = jnp.zeros((1, P), jnp.float32)
    for c, g in enumerate((g0, g1, g2, g3)):
        d = locd[c:c + 1, :] - g
        a = jnp.abs(d)
        sl1 = sl1 + jnp.where(a < 1.0, 0.5 * d * d, a - 0.5)
    loss_l_2d = jnp.sum(jnp.where(pos, sl1, 0.0), axis=1, keepdims=True)

    ones_c = jnp.ones((1, NUM_CLASSES), jnp.float32)
    m = jnp.max(conf, axis=0, keepdims=True)
    e = jnp.exp(conf - m)
    s = jax.lax.dot_general(
        ones_c, e, (((1,), (0,)), ((), ())),
        preferred_element_type=jnp.float32)
    mls = m + jnp.log(s)
    lp = jnp.clip(conf - mls, CLIP_LO, CLIP_HI)
    csub = jax.lax.broadcasted_iota(jnp.int32, (NUM_CLASSES, P), 0)
    eq_c = csub == conf_t
    gathered = jnp.sum(jnp.where(eq_c, conf, 0.0), axis=0, keepdims=True)
    lpu_tgt = gathered - mls
    lp_tgt = jnp.clip(lpu_tgt, CLIP_LO, CLIP_HI)
    lp_all = jax.lax.dot_general(
        ones_c, lp, (((1,), (0,)), ((), ())),
        preferred_element_type=jnp.float32)
    mloss = jnp.where(pos, 0.0, -lpu_tgt)
    eps_o = EPS / (NUM_CLASSES - 1)
    row_loss = -(eps_o * lp_all + (1.0 - EPS - eps_o) * lp_tgt)
    rl_neg = jnp.where(pos, 0.0, row_loss)
    rl_pos_2d = jnp.sum(jnp.where(pos, row_loss, 0.0), axis=1, keepdims=True)
    npos_2d = jnp.sum(pos.astype(jnp.float32), axis=1, keepdims=True)

    ml_ref[img] = mloss
    rl_ref[img] = rl_neg
    li = jax.lax.broadcasted_iota(jnp.int32, (1, 128), 1)
    stats = (jnp.where(li == 0, loss_l_2d, 0.0)
             + jnp.where(li == 1, npos_2d, 0.0)
             + jnp.where(li == 2, rl_pos_2d, 0.0))
    st_ref[img] = stats


def _phase_b(ml_ref, rl_ref, st_ref, out_l, out_c):
    ml = ml_ref[...][:, 0, :]
    rl = rl_ref[...][:, 0, :]
    st = st_ref[...][:, 0, :]
    B, P = ml.shape

    ll_tot = jnp.sum(st[:, 0:1], axis=0, keepdims=True)
    npos = st[:, 1:2]
    rp_tot = jnp.sum(st[:, 2:3], axis=0, keepdims=True)
    n_tot = jnp.maximum(jnp.sum(npos, axis=0, keepdims=True), 1.0)

    k = jnp.minimum(NEGPOS_RATIO * npos.astype(jnp.int32), P - 1)
    bits = jax.lax.bitcast_convert_type(ml, jnp.int32)

    def vstep(_, lh):
        lo, hi = lh
        mid = lo + (hi - lo) // 2
        cnt = jnp.sum(jnp.where(bits > mid, 1, 0), axis=1, keepdims=True)
        p = cnt < k
        return jnp.where(p, lo, mid + 1), jnp.where(p, mid, hi)

    zc = jnp.zeros((B, 1), jnp.int32)
    lo, hi = jax.lax.fori_loop(
        0, 31, vstep, (zc, jnp.full((B, 1), 0x7F800000, jnp.int32)))
    tb = hi
    gcnt = jnp.sum(jnp.where(bits > tb, 1, 0), axis=1, keepdims=True)
    need = k - gcnt
    equal = bits == tb
    lane = jax.lax.broadcasted_iota(jnp.int32, (B, P), 1)

    def istep(_, lh):
        lo2, hi2 = lh
        mid = lo2 + (hi2 - lo2) // 2
        cnt = jnp.sum(jnp.where(equal & (lane < mid), 1, 0),
                      axis=1, keepdims=True)
        p = cnt >= need
        return jnp.where(p, lo2, mid + 1), jnp.where(p, mid, hi2)

    j, _ = jax.lax.fori_loop(0, 14, istep, (zc, jnp.full((B, 1), P, jnp.int32)))
    neg = (bits > tb) | (equal & (lane < j))
    lc_neg = jnp.sum(jnp.sum(jnp.where(neg, rl, 0.0), axis=1, keepdims=True),
                     axis=0, keepdims=True)

    out_l[...] = ll_tot / n_tot
    out_c[...] = (rp_tot + lc_neg) / n_tot


def kernel(loc_data, conf_data, priors, targets):
    num, num_priors, _ = loc_data.shape
    conf_cm = jnp.transpose(conf_data, (0, 2, 1))
    loc_cm = jnp.transpose(loc_data, (0, 2, 1))
    pri_cm = jnp.transpose(priors[:num_priors], (1, 0))

    ipg = 4
    ml, rl, st = pl.pallas_call(
        _phase_a,
        grid=(num // ipg,),
        in_specs=[
            pl.BlockSpec((ipg, NUM_CLASSES, num_priors), lambda i: (i, 0, 0)),
            pl.BlockSpec((ipg, 4, num_priors), lambda i: (i, 0, 0)),
            pl.BlockSpec((4, num_priors), lambda i: (0, 0)),
            pl.BlockSpec((ipg, targets.shape[1], targets.shape[2]),
                         lambda i: (i, 0, 0)),
        ],
        out_specs=[
            pl.BlockSpec((ipg, 1, num_priors), lambda i: (i, 0, 0)),
            pl.BlockSpec((ipg, 1, num_priors), lambda i: (i, 0, 0)),
            pl.BlockSpec((ipg, 1, 128), lambda i: (i, 0, 0)),
        ],
        out_shape=[
            jax.ShapeDtypeStruct((num, 1, num_priors), jnp.float32),
            jax.ShapeDtypeStruct((num, 1, num_priors), jnp.float32),
            jax.ShapeDtypeStruct((num, 1, 128), jnp.float32),
        ],
        compiler_params=pltpu.CompilerParams(
            dimension_semantics=("parallel",)),
    )(conf_cm, loc_cm, pri_cm, targets)

    out_l, out_c = pl.pallas_call(
        _phase_b,
        out_shape=[jax.ShapeDtypeStruct((1, 1), jnp.float32)] * 2,
    )(ml, rl, st)
    return (out_l[0, 0], out_c[0, 0])
